# baseline (speedup 1.0000x reference)
import functools

import jax
import jax.numpy as jnp
from jax import lax
from jax.experimental import pallas as pl
from jax.experimental.pallas import tpu as pltpu
from jax.experimental.pallas import tpu_sc as plsc

_LANES = 16
_NUM_CORES = 2
_NUM_SUBCORES = 16
_NUM_WORKERS = _NUM_CORES * _NUM_SUBCORES


def _mean_body(att_ref, avg_ref):
    att = att_ref[...]
    avg_ref[...] = jnp.sum(att, axis=0) * (1.0 / att.shape[0])


def _head_mean_one(att_b):
    H, N, _ = att_b.shape
    BI = 64
    return pl.pallas_call(
        _mean_body,
        grid=(N // BI,),
        in_specs=[pl.BlockSpec((H, BI, N), lambda i: (0, i, 0))],
        out_specs=pl.BlockSpec((BI, N), lambda i: (i, 0)),
        out_shape=jax.ShapeDtypeStruct((N, N), jnp.float32),
    )(att_b)


def _make_sc_bag(R, N, T):
    rows_w = R // _NUM_WORKERS
    n_chunks = N // _LANES
    mesh = plsc.VectorSubcoreMesh(core_axis_name="c", subcore_axis_name="s")

    @functools.partial(
        pl.kernel,
        mesh=mesh,
        out_type=jax.ShapeDtypeStruct((R, T * _LANES), jnp.float32),
        scratch_types=[
            pltpu.VMEM((rows_w, N), jnp.float32),
            pltpu.VMEM((rows_w, N), jnp.int32),
            pltpu.VMEM((rows_w, T * _LANES), jnp.float32),
        ],
    )
    def sc_bag(avg_hbm, etT_hbm, bins_hbm, avg_v, et_v, bins_v):
        wid = lax.axis_index("s") * _NUM_CORES + lax.axis_index("c")
        base = wid * rows_w
        pltpu.sync_copy(avg_hbm.at[pl.ds(base, rows_w)], avg_v)
        pltpu.sync_copy(etT_hbm.at[pl.ds(base, rows_w)], et_v)
        zero16 = jnp.zeros((_LANES,), jnp.float32)

        def row_body(r, carry):
            acc = [zero16] * T
            for k in range(n_chunks):
                w = avg_v[r, pl.ds(k * _LANES, _LANES)]
                tv = et_v[r, pl.ds(k * _LANES, _LANES)]
                for t in range(T):
                    acc[t] = acc[t] + jnp.where(tv == t, w, 0.0)
            for t in range(T):
                bins_v[r, pl.ds(t * _LANES, _LANES)] = acc[t]
            return carry

        lax.fori_loop(0, rows_w, row_body, 0)
        pltpu.sync_copy(bins_v, bins_hbm.at[pl.ds(base, rows_w)])

    return sc_bag


def _combine_body(bins_ref, w_ref, out_ref):
    out_ref[...] = jax.lax.dot_general(
        bins_ref[...], w_ref[...], (((1,), (0,)), ((), ())),
        preferred_element_type=jnp.float32,
        precision=jax.lax.Precision.HIGHEST)


def _combine(bins, w_rep, R):
    TL, D = w_rep.shape
    return pl.pallas_call(
        _combine_body,
        in_specs=[
            pl.BlockSpec((R, TL), lambda: (0, 0)),
            pl.BlockSpec((TL, D), lambda: (0, 0)),
        ],
        out_specs=pl.BlockSpec((R, D), lambda: (0, 0)),
        out_shape=jax.ShapeDtypeStruct((R, D), jnp.float32),
    )(bins, w_rep)


def kernel(attention_weights, edge_type_matrix, embedding_table):
    B, H, N, _ = attention_weights.shape
    T, D = embedding_table.shape
    R = B * N
    etT = jnp.swapaxes(edge_type_matrix.astype(jnp.int32), 1, 2)
    w_rep = jnp.repeat(embedding_table, _LANES, axis=0)
    sc_bag = _make_sc_bag(N, N, T)
    bins = []
    for b in range(B):
        avg_b = _head_mean_one(attention_weights[b])
        bins.append(sc_bag(avg_b, etT[b]))
    out = _combine(jnp.concatenate(bins, axis=0), w_rep, R)
    return out.reshape(B, N, D)


# speedup vs baseline: 1.5612x; 1.5612x over previous
import functools

import jax
import jax.numpy as jnp
from jax import lax
from jax.experimental import pallas as pl
from jax.experimental.pallas import tpu as pltpu
from jax.experimental.pallas import tpu_sc as plsc

_LANES = 16
_NUM_CORES = 2
_NUM_SUBCORES = 16
_NUM_WORKERS = _NUM_CORES * _NUM_SUBCORES


def _mean_body(att_ref, avg_ref):
    att = att_ref[0]
    avg_ref[0] = jnp.sum(att, axis=0) * (1.0 / att.shape[0])


def _head_mean(attention_weights):
    B, H, N, _ = attention_weights.shape
    BI = 64
    return pl.pallas_call(
        _mean_body,
        grid=(B, N // BI),
        in_specs=[pl.BlockSpec((1, H, BI, N), lambda b, i: (b, 0, i, 0))],
        out_specs=pl.BlockSpec((1, BI, N), lambda b, i: (b, i, 0)),
        out_shape=jax.ShapeDtypeStruct((B, N, N), jnp.float32),
    )(attention_weights)


def _make_sc_bag(R, N, T):
    rows_w = R // _NUM_WORKERS
    n_chunks = N // _LANES
    mesh = plsc.VectorSubcoreMesh(core_axis_name="c", subcore_axis_name="s")

    @functools.partial(
        pl.kernel,
        mesh=mesh,
        out_type=jax.ShapeDtypeStruct((R, T * _LANES), jnp.float32),
        scratch_types=[
            pltpu.VMEM((rows_w, N), jnp.float32),
            pltpu.VMEM((rows_w, N), jnp.int32),
            pltpu.VMEM((rows_w, T * _LANES), jnp.float32),
        ],
    )
    def sc_bag(avg_hbm, etT_hbm, bins_hbm, avg_v, et_v, bins_v):
        wid = lax.axis_index("s") * _NUM_CORES + lax.axis_index("c")
        base = wid * rows_w
        pltpu.sync_copy(avg_hbm.at[pl.ds(base, rows_w)], avg_v)
        pltpu.sync_copy(etT_hbm.at[pl.ds(base, rows_w)], et_v)
        zero16 = jnp.zeros((_LANES,), jnp.float32)

        def row_body(r, carry):
            acc = [zero16] * T
            for k in range(n_chunks):
                w = avg_v[r, pl.ds(k * _LANES, _LANES)]
                tv = et_v[r, pl.ds(k * _LANES, _LANES)]
                for t in range(T):
                    acc[t] = acc[t] + jnp.where(tv == t, w, 0.0)
            for t in range(T):
                bins_v[r, pl.ds(t * _LANES, _LANES)] = acc[t]
            return carry

        lax.fori_loop(0, rows_w, row_body, 0)
        pltpu.sync_copy(bins_v, bins_hbm.at[pl.ds(base, rows_w)])

    return sc_bag


def _combine_body(bins_ref, w_ref, out_ref):
    out_ref[...] = jax.lax.dot_general(
        bins_ref[...], w_ref[...], (((1,), (0,)), ((), ())),
        preferred_element_type=jnp.float32,
        precision=jax.lax.Precision.HIGHEST)


def _combine(bins, w_rep, R):
    TL, D = w_rep.shape
    return pl.pallas_call(
        _combine_body,
        in_specs=[
            pl.BlockSpec((R, TL), lambda: (0, 0)),
            pl.BlockSpec((TL, D), lambda: (0, 0)),
        ],
        out_specs=pl.BlockSpec((R, D), lambda: (0, 0)),
        out_shape=jax.ShapeDtypeStruct((R, D), jnp.float32),
    )(bins, w_rep)


def kernel(attention_weights, edge_type_matrix, embedding_table):
    B, H, N, _ = attention_weights.shape
    T, D = embedding_table.shape
    R = B * N
    etT = jnp.swapaxes(edge_type_matrix.astype(jnp.int32), 1, 2)
    w_rep = jnp.repeat(embedding_table, _LANES, axis=0)
    avg = _head_mean(attention_weights)
    bins = _make_sc_bag(R, N, T)(avg.reshape(R, N), etT.reshape(R, N))
    out = _combine(bins, w_rep, R)
    return out.reshape(B, N, D)


# speedup vs baseline: 1.7533x; 1.1230x over previous
import functools

import jax
import jax.numpy as jnp
from jax import lax
from jax.experimental import pallas as pl
from jax.experimental.pallas import tpu as pltpu
from jax.experimental.pallas import tpu_sc as plsc

_LANES = 16
_NUM_CORES = 2
_NUM_SUBCORES = 16
_NUM_WORKERS = _NUM_CORES * _NUM_SUBCORES


def _mean_body(att_ref, et_ref, avg_ref, etT_ref):
    att = att_ref[0]
    avg_ref[0] = jnp.sum(att, axis=0) * (1.0 / att.shape[0])
    etT_ref[0] = jnp.swapaxes(et_ref[0], 0, 1)


def _head_mean(attention_weights, et_i32):
    B, H, N, _ = attention_weights.shape
    BI = 128
    return pl.pallas_call(
        _mean_body,
        grid=(B, N // BI),
        in_specs=[
            pl.BlockSpec((1, H, BI, N), lambda b, i: (b, 0, i, 0)),
            pl.BlockSpec((1, N, BI), lambda b, i: (b, 0, i)),
        ],
        out_specs=[
            pl.BlockSpec((1, BI, N), lambda b, i: (b, i, 0)),
            pl.BlockSpec((1, BI, N), lambda b, i: (b, i, 0)),
        ],
        out_shape=[
            jax.ShapeDtypeStruct((B, N, N), jnp.float32),
            jax.ShapeDtypeStruct((B, N, N), jnp.int32),
        ],
    )(attention_weights, et_i32)


def _make_sc_bag(R, N, T):
    rows_w = R // _NUM_WORKERS
    n_chunks = N // _LANES
    mesh = plsc.VectorSubcoreMesh(core_axis_name="c", subcore_axis_name="s")

    @functools.partial(
        pl.kernel,
        mesh=mesh,
        out_type=jax.ShapeDtypeStruct((R, T * _LANES), jnp.float32),
        scratch_types=[
            pltpu.VMEM((rows_w, N), jnp.float32),
            pltpu.VMEM((rows_w, N), jnp.int32),
            pltpu.VMEM((rows_w, T * _LANES), jnp.float32),
        ],
    )
    def sc_bag(avg_hbm, etT_hbm, bins_hbm, avg_v, et_v, bins_v):
        wid = lax.axis_index("s") * _NUM_CORES + lax.axis_index("c")
        base = wid * rows_w
        pltpu.sync_copy(avg_hbm.at[pl.ds(base, rows_w)], avg_v)
        pltpu.sync_copy(etT_hbm.at[pl.ds(base, rows_w)], et_v)
        zero16 = jnp.zeros((_LANES,), jnp.float32)

        def row_body(r, carry):
            acc = [zero16] * T
            for k in range(n_chunks):
                w = avg_v[r, pl.ds(k * _LANES, _LANES)]
                tv = et_v[r, pl.ds(k * _LANES, _LANES)]
                for t in range(T):
                    acc[t] = acc[t] + jnp.where(tv == t, w, 0.0)
            for t in range(T):
                bins_v[r, pl.ds(t * _LANES, _LANES)] = acc[t]
            return carry

        lax.fori_loop(0, rows_w, row_body, 0)
        pltpu.sync_copy(bins_v, bins_hbm.at[pl.ds(base, rows_w)])

    return sc_bag


def _combine_body(bins_ref, w_ref, out_ref):
    out_ref[...] = jax.lax.dot_general(
        bins_ref[...], w_ref[...], (((1,), (0,)), ((), ())),
        preferred_element_type=jnp.float32,
        precision=jax.lax.Precision.HIGHEST)


def _combine(bins, w_rep, R):
    TL, D = w_rep.shape
    return pl.pallas_call(
        _combine_body,
        in_specs=[
            pl.BlockSpec((R, TL), lambda: (0, 0)),
            pl.BlockSpec((TL, D), lambda: (0, 0)),
        ],
        out_specs=pl.BlockSpec((R, D), lambda: (0, 0)),
        out_shape=jax.ShapeDtypeStruct((R, D), jnp.float32),
    )(bins, w_rep)


def kernel(attention_weights, edge_type_matrix, embedding_table):
    B, H, N, _ = attention_weights.shape
    T, D = embedding_table.shape
    R = B * N
    w_rep = jnp.repeat(embedding_table, _LANES, axis=0)
    avg, etT = _head_mean(attention_weights, edge_type_matrix.astype(jnp.int32))
    bins = _make_sc_bag(R, N, T)(avg.reshape(R, N), etT.reshape(R, N))
    out = _combine(bins, w_rep, R)
    return out.reshape(B, N, D)
